# baseline (speedup 1.0000x reference)
import functools

import jax
import jax.numpy as jnp
from jax import lax
from jax.experimental import pallas as pl
from jax.experimental.pallas import tpu as pltpu
from jax.experimental.pallas import tpu_sc as plsc

D_MODEL = 64
ROWS = 4096
COLS = 200
B = ROWS * COLS
NUM_WORKERS = 32
BPW = B // NUM_WORKERS
GSZ = 128
NG = BPW // GSZ
SCALE = 8.0
LANES = 16


def _embed_body(x_hbm, lut_hbm, out_hbm, idx_v, rows_v, gsem):
    c = lax.axis_index("c")
    s = lax.axis_index("s")
    wid = s * 2 + c

    pltpu.sync_copy(x_hbm.at[wid], idx_v)

    def step(j, carry):
        pltpu.async_copy(lut_hbm.at[idx_v.at[j]], rows_v, gsem).wait()

        def scale_row(r, c2):
            for q in range(4):
                rows_v[r, pl.ds(LANES * q, LANES)] = (
                    rows_v[r, pl.ds(LANES * q, LANES)] * SCALE
                )
            return c2

        lax.fori_loop(0, GSZ, scale_row, 0, unroll=2)

        pltpu.sync_copy(rows_v, out_hbm.at[pl.ds(wid * BPW + j * GSZ, GSZ)])
        return carry

    lax.fori_loop(0, NG, step, 0)


@jax.jit
def _embed(x2d, lut):
    mesh = plsc.VectorSubcoreMesh(core_axis_name="c", subcore_axis_name="s")
    kern = functools.partial(
        pl.kernel,
        out_type=jax.ShapeDtypeStruct((B, D_MODEL), jnp.float32),
        mesh=mesh,
        compiler_params=pltpu.CompilerParams(use_tc_tiling_on_sc=False),
        scratch_types=[
            pltpu.VMEM((NG, GSZ), jnp.int32),
            pltpu.VMEM((GSZ, D_MODEL), jnp.float32),
            pltpu.SemaphoreType.DMA,
        ],
    )(_embed_body)
    return kern(x2d, lut)


def kernel(x, lut):
    x2d = x.astype(jnp.int32).reshape(NUM_WORKERS, NG, GSZ)
    out = _embed(x2d, lut)
    return out.reshape(ROWS, COLS, D_MODEL)


# speedup vs baseline: 1.1600x; 1.1600x over previous
import functools

import jax
import jax.numpy as jnp
from jax import lax
from jax.experimental import pallas as pl
from jax.experimental.pallas import tpu as pltpu
from jax.experimental.pallas import tpu_sc as plsc

D_MODEL = 64
ROWS = 4096
COLS = 200
B = ROWS * COLS
NUM_WORKERS = 32
BPW = B // NUM_WORKERS
GSZ = 128
NG = BPW // GSZ
NBUF = 4
LEAD = 2
SCALE = 8.0
LANES = 16


def _embed_body(x_hbm, lut_hbm, out_hbm, idx_v, rows_v, gsems, ssems):
    c = lax.axis_index("c")
    s = lax.axis_index("s")
    wid = s * 2 + c
    out_base = wid * BPW

    pltpu.sync_copy(x_hbm.at[wid], idx_v)

    def start_gather(jj, b):
        pltpu.async_copy(lut_hbm.at[idx_v.at[jj]], rows_v.at[b], gsems[b])

    def wait_gather(b):
        pltpu.make_async_copy(lut_hbm.at[idx_v.at[0]], rows_v.at[b],
                              gsems[b]).wait()

    def start_store(j, b):
        pltpu.async_copy(rows_v.at[b], out_hbm.at[pl.ds(out_base + j * GSZ, GSZ)],
                         ssems[b])

    def wait_store(b):
        pltpu.make_async_copy(rows_v.at[b],
                              out_hbm.at[pl.ds(out_base, GSZ)], ssems[b]).wait()

    def scale_chunk(b):
        def scale_row(r, carry):
            for q in range(4):
                rows_v[b, r, pl.ds(LANES * q, LANES)] = (
                    rows_v[b, r, pl.ds(LANES * q, LANES)] * SCALE
                )
            return carry

        lax.fori_loop(0, GSZ, scale_row, 0, unroll=4)

    start_gather(0, 0)
    start_gather(1, 1)

    for b in range(NBUF):
        bg = (b + LEAD) % NBUF
        if b >= LEAD:
            wait_store(bg)
        start_gather(b + LEAD, bg)
        wait_gather(b)
        scale_chunk(b)
        start_store(b, b)

    def ring_iter(i, carry):
        j0 = i * NBUF
        for b in range(NBUF):
            bg = (b + LEAD) % NBUF
            wait_store(bg)
            start_gather(j0 + b + LEAD, bg)
            wait_gather(b)
            scale_chunk(b)
            start_store(j0 + b, b)
        return carry

    lax.fori_loop(1, NG // NBUF - 1, ring_iter, 0)

    j0 = NG - NBUF
    for b in range(NBUF):
        jj = j0 + b + LEAD
        if jj < NG:
            bg = (b + LEAD) % NBUF
            wait_store(bg)
            start_gather(jj, bg)
        wait_gather(b)
        scale_chunk(b)
        start_store(j0 + b, b)

    for b in range(NBUF):
        wait_store(b)


@jax.jit
def _embed(x2d, lut):
    mesh = plsc.VectorSubcoreMesh(core_axis_name="c", subcore_axis_name="s")
    kern = functools.partial(
        pl.kernel,
        out_type=jax.ShapeDtypeStruct((B, D_MODEL), jnp.float32),
        mesh=mesh,
        compiler_params=pltpu.CompilerParams(use_tc_tiling_on_sc=False),
        scratch_types=[
            pltpu.VMEM((NG, GSZ), jnp.int32),
            pltpu.VMEM((NBUF, GSZ, D_MODEL), jnp.float32),
            [pltpu.SemaphoreType.DMA] * NBUF,
            [pltpu.SemaphoreType.DMA] * NBUF,
        ],
    )(_embed_body)
    return kern(x2d, lut)


def kernel(x, lut):
    x2d = x.astype(jnp.int32).reshape(NUM_WORKERS, NG, GSZ)
    out = _embed(x2d, lut)
    return out.reshape(ROWS, COLS, D_MODEL)
